# baseline (speedup 1.0000x reference)
import functools

import jax
import jax.numpy as jnp
from jax.experimental import pallas as pl
from jax.experimental.pallas import tpu as pltpu

N_SPECIES = 1024
N_REACTIONS = 65536
S_BLOCK = 64
C_CHUNK = 2048


def _rates_chunk(c, rw_ref, sw_ref, la, al, be, ga, cc, fc, t, cr, fuv):
    C = C_CHUNK
    lo_r = c * C
    s0 = max(0, (2 * c - 1) * C)
    e0 = (2 * c + 2) * C
    w = e0 - s0
    rw = rw_ref[0:1, s0:e0]
    sw = sw_ref[0:1, s0:e0]

    rates0 = (al * jnp.exp(be * jnp.log(t / 300.0) - ga / t)
              + cc * cr + fc * fuv)

    iota32 = jax.lax.broadcasted_iota(jnp.int32, (32, w), 0)
    oh_lo = jnp.where(iota32 == (sw & 31), 1.0, 0.0)
    cols = jax.lax.dot_general(la, oh_lo, (((1,), (0,)), ((), ())),
                               preferred_element_type=jnp.float32)
    v = jnp.sum(jnp.where(iota32 == (sw >> 5), cols, 0.0),
                axis=0, keepdims=True)

    off = rw - lo_r
    hi_rows = C >> 5
    iota_hi = jax.lax.broadcasted_iota(jnp.int32, (hi_rows, w), 0)
    bv = jnp.where(iota_hi == (off >> 5), v, 0.0)
    oh_lo2 = jnp.where(iota32 == (off & 31), 1.0, 0.0)
    g = jax.lax.dot_general(bv, oh_lo2, (((1,), (1,)), ((), ())),
                            preferred_element_type=jnp.float32)

    e = jnp.exp(g)
    tiled = jnp.tile(e, (1, hi_rows))
    lane = jax.lax.broadcasted_iota(jnp.int32, (hi_rows, C), 1)
    rows = jax.lax.broadcasted_iota(jnp.int32, (hi_rows, C), 0)
    flat = jnp.sum(jnp.where(rows == (lane >> 5), tiled, 0.0),
                   axis=0, keepdims=True)

    return rates0 * flat


def _fused_kernel(t_ref, cr_ref, fuv_ref, ab_ref, al_ref, be_ref, ga_ref,
                  cc_ref, fc_ref, rw_ref, sw_ref, inc_ref, out_ref,
                  rates_ref):
    k = pl.program_id(0)

    @pl.when(k == 0)
    def _build_rates():
        t = t_ref[0, 0]
        cr = cr_ref[0, 0]
        fuv = fuv_ref[0, 0]
        la = jnp.log(ab_ref[:, :])
        C = C_CHUNK
        for c in range(N_REACTIONS // C):
            sl = slice(c * C, (c + 1) * C)
            rates_ref[0:1, sl] = _rates_chunk(
                c, rw_ref, sw_ref, la,
                al_ref[0:1, sl], be_ref[0:1, sl], ga_ref[0:1, sl],
                cc_ref[0:1, sl], fc_ref[0:1, sl], t, cr, fuv)

    out_ref[:, :] = jax.lax.dot_general(
        inc_ref[:, :], rates_ref[0:1, :], (((1,), (1,)), ((), ())),
        preferred_element_type=jnp.float32)


def kernel(abundances, temperature, cr_rate, fuv_rate, incidence, alpha, beta,
           gamma, cr_coef, fuv_coef, reac_idx, species_idx):
    n_pairs = reac_idx.shape[0]
    deficit = 2 * N_REACTIONS - n_pairs
    if deficit > C_CHUNK:
        raise ValueError("pair-list deficit exceeds one reaction chunk")

    l_pad = 2 * N_REACTIONS
    pad = l_pad - n_pairs
    rw = jnp.pad(reac_idx.astype(jnp.int32), (0, pad),
                 constant_values=N_REACTIONS).reshape(1, l_pad)
    sw = jnp.pad(species_idx.astype(jnp.int32), (0, pad),
                 constant_values=0).reshape(1, l_pad)

    row = lambda x: x.reshape(1, -1)
    scl = lambda x: x.reshape(1, 1).astype(jnp.float32)
    whole = lambda shape: pl.BlockSpec(shape, lambda k: (0, 0))

    out = pl.pallas_call(
        _fused_kernel,
        grid=(N_SPECIES // S_BLOCK,),
        in_specs=[
            whole((1, 1)), whole((1, 1)), whole((1, 1)),
            whole((32, 32)),
            whole((1, N_REACTIONS)), whole((1, N_REACTIONS)),
            whole((1, N_REACTIONS)), whole((1, N_REACTIONS)),
            whole((1, N_REACTIONS)),
            whole((1, l_pad)), whole((1, l_pad)),
            pl.BlockSpec((S_BLOCK, N_REACTIONS), lambda k: (k, 0)),
        ],
        out_specs=pl.BlockSpec((S_BLOCK, 1), lambda k: (k, 0)),
        out_shape=jax.ShapeDtypeStruct((N_SPECIES, 1), jnp.float32),
        scratch_shapes=[pltpu.VMEM((1, N_REACTIONS), jnp.float32)],
        compiler_params=pltpu.CompilerParams(
            dimension_semantics=("arbitrary",),
        ),
    )(scl(temperature), scl(cr_rate), scl(fuv_rate),
      abundances.reshape(32, 32),
      row(alpha), row(beta), row(gamma), row(cr_coef), row(fuv_coef),
      rw, sw, incidence)
    return out.reshape(N_SPECIES)
